# baseline (speedup 1.0000x reference)
import functools

import numpy as np
import jax
import jax.numpy as jnp
from jax import lax
from jax.experimental import pallas as pl
from jax.experimental.pallas import tpu as pltpu
from jax.experimental.pallas import tpu_sc as plsc

N_NODES = 10000
N_EDGES = 320000
D = 128

NC = 2
NS = 16
NW = NC * NS

B = 128
NB = 80
EPT = NB * B
EPAD = EPT * NW

NPAD = 10240
RPT = NPAD // NS

_MESH = plsc.VectorSubcoreMesh(core_axis_name="c", subcore_axis_name="s")

_Q32 = np.concatenate([np.arange(16) * 2, np.arange(16) * 2 + 1])
QPERM = np.concatenate([g * 32 + _Q32 for g in range(D // 32)])


def _deg_body(dst_hbm, ones_hbm, zeros_hbm, out_hbm, dstv, onesv, acc):
    c = lax.axis_index("c")
    s = lax.axis_index("s")
    wid = c * NS + s

    pltpu.sync_copy(ones_hbm, onesv)

    for r in range(RPT // B):
        pltpu.sync_copy(zeros_hbm, acc.at[pl.ds(s * RPT + r * B, B)])
    plsc.subcore_barrier()

    def body(jc, carry):
        base = pl.multiple_of(wid * NB + jc * 8, 8)
        pltpu.sync_copy(dst_hbm.at[pl.ds(base, 8)], dstv)
        for jj in range(8):
            pltpu.sync_copy(onesv, acc.at[dstv.at[jj]], add=True)
        return carry

    lax.fori_loop(0, NB // 8, body, 0)
    plsc.subcore_barrier()

    base = c * NPAD + s * RPT
    pltpu.sync_copy(acc.at[pl.ds(s * RPT, RPT)], out_hbm.at[pl.ds(base, RPT)])


@functools.partial(
    pl.kernel,
    mesh=_MESH,
    out_type=jax.ShapeDtypeStruct((NC * NPAD, D), jnp.float32),
    scratch_types=[
        pltpu.VMEM((8, B), jnp.int32),
        pltpu.VMEM((B, D), jnp.float32),
        pltpu.VMEM_SHARED((NPAD, D), jnp.float32),
    ],
)
def _deg_kernel(dst_hbm, ones_hbm, zeros_hbm, out_hbm, dstv, onesv, acc):
    _deg_body(dst_hbm, ones_hbm, zeros_hbm, out_hbm, dstv, onesv, acc)


_CH = 8
_NCHUNK = NB // _CH


def _scat_body(y_hbm, src_hbm, dst_hbm, zeros_hbm, out_hbm,
               srcv, dstv, rows0, rows1, bfb, gsem0, gsem1, ssem0, ssem1,
               acc):
    c = lax.axis_index("c")
    s = lax.axis_index("s")
    wid = c * NS + s

    for r in range(RPT // B):
        pltpu.sync_copy(zeros_hbm, acc.at[pl.ds(s * RPT + r * B, B)])
    plsc.subcore_barrier()

    rows = (rows0, rows1)
    gsems = (gsem0, gsem1)
    ssems = (ssem0, ssem1)

    def chunk(jc, carry):
        base = pl.multiple_of(wid * NB + jc * _CH, _CH)
        pltpu.sync_copy(src_hbm.at[pl.ds(base, _CH)], srcv)
        pltpu.sync_copy(dst_hbm.at[pl.ds(base, _CH)], dstv)
        g = [pltpu.async_copy(y_hbm.at[srcv.at[0]], rows[0], gsems[0]),
             pltpu.async_copy(y_hbm.at[srcv.at[1]], rows[1], gsems[1])]
        sc = [None, None]
        for jj in range(_CH):
            b = jj % 2
            g[b].wait()
            sc[b] = pltpu.async_copy(rows[b], acc.at[dstv.at[jj]], ssems[b],
                                     add=True)
            if jj + 2 < _CH:
                sc[b].wait()
                g[b] = pltpu.async_copy(y_hbm.at[srcv.at[jj + 2]], rows[b],
                                        gsems[b])
        sc[0].wait()
        sc[1].wait()
        return carry

    lax.fori_loop(0, _NCHUNK, chunk, 0)
    plsc.subcore_barrier()

    base = (c * NPAD + s * RPT) // 2
    for r in range(RPT // B):
        pltpu.sync_copy(acc.at[pl.ds(s * RPT + r * B, B)], rows0)

        c7 = jnp.full((16,), 0x7FFF, jnp.int32)
        c1 = jnp.full((16,), 1, jnp.int32)
        c16 = jnp.full((16,), 16, jnp.int32)
        mhi = jnp.full((16,), -65536, jnp.int32)

        def rnd(v):
            vi = plsc.bitcast(v, jnp.int32)
            return vi + c7 + (lax.shift_right_logical(vi, c16) & c1)

        def conv(rr, carry):
            frow = rows0.at[rr]
            for g in range(D // 32):
                a = frow[pl.ds(g * 32, 16)]
                b_ = frow[pl.ds(g * 32 + 16, 16)]
                packed = (lax.shift_right_logical(rnd(a), c16)
                          | (rnd(b_) & mhi))
                bfb[rr // 2, pl.ds((rr % 2) * (D // 2) + g * 16, 16)] = packed
            return carry

        lax.fori_loop(0, B, conv, 0)
        pltpu.sync_copy(
            bfb,
            out_hbm.at[pl.ds(pl.multiple_of(base + r * (B // 2), B // 2),
                             B // 2)])


@functools.partial(
    pl.kernel,
    mesh=_MESH,
    compiler_params=pltpu.CompilerParams(needs_layout_passes=False),
    out_type=jax.ShapeDtypeStruct((NC * NPAD // 2, D), jnp.int32),
    scratch_types=[
        pltpu.VMEM((_CH, B), jnp.int32),
        pltpu.VMEM((_CH, B), jnp.int32),
        pltpu.VMEM((B, D), jnp.float32),
        pltpu.VMEM((B, D), jnp.float32),
        pltpu.VMEM((B // 2, D), jnp.int32),
        pltpu.SemaphoreType.DMA,
        pltpu.SemaphoreType.DMA,
        pltpu.SemaphoreType.DMA,
        pltpu.SemaphoreType.DMA,
        pltpu.VMEM_SHARED((NPAD, D), jnp.float32),
    ],
)
def _scat_kernel(y_hbm, src_hbm, dst_hbm, zeros_hbm, out_hbm,
                 srcv, dstv, rows0, rows1, bfb, gsem0, gsem1, ssem0, ssem1,
                 acc):
    _scat_body(y_hbm, src_hbm, dst_hbm, zeros_hbm, out_hbm,
               srcv, dstv, rows0, rows1, bfb, gsem0, gsem1, ssem0, ssem1,
               acc)


_BLK = 512
_GRID = NPAD // _BLK


def _k1_body(x_ref, d0_ref, d1_ref, w_ref, wq_ref, y_ref, yq_ref, dbc_ref):
    deg = d0_ref[...] + d1_ref[...] + 1.0
    d = lax.rsqrt(deg)
    x = x_ref[...]
    y_ref[...] = jnp.dot(x, w_ref[...],
                         preferred_element_type=jnp.float32) * d
    yq_ref[...] = jnp.dot(x, wq_ref[...],
                          preferred_element_type=jnp.float32) * d
    dbc_ref[...] = d


def _tc_scale_matmul(x_pad, deg0, deg1, W1, W1q):
    return pl.pallas_call(
        _k1_body,
        grid=(_GRID,),
        in_specs=[
            pl.BlockSpec((_BLK, D), lambda i: (i, 0)),
            pl.BlockSpec((_BLK, D), lambda i: (i, 0)),
            pl.BlockSpec((_BLK, D), lambda i: (i, 0)),
            pl.BlockSpec((D, D), lambda i: (0, 0)),
            pl.BlockSpec((D, D), lambda i: (0, 0)),
        ],
        out_specs=[
            pl.BlockSpec((_BLK, D), lambda i: (i, 0)),
            pl.BlockSpec((_BLK, D), lambda i: (i, 0)),
            pl.BlockSpec((_BLK, D), lambda i: (i, 0)),
        ],
        out_shape=[
            jax.ShapeDtypeStruct((NPAD, D), jnp.float32),
            jax.ShapeDtypeStruct((NPAD, D), jnp.float32),
            jax.ShapeDtypeStruct((NPAD, D), jnp.float32),
        ],
    )(x_pad, deg0, deg1, W1, W1q)


def _k2_body(s0_ref, s1_ref, y_ref, dbc_ref, b_ref, w_ref, wq_ref,
             h_ref, y2_ref, y2q_ref):
    dbc = dbc_ref[...]
    s = (s0_ref[...].astype(jnp.float32) + s1_ref[...].astype(jnp.float32))
    h = jnp.maximum(dbc * (s + y_ref[...]) + b_ref[...], 0.0)
    h_ref[...] = h
    y2_ref[...] = jnp.dot(h, w_ref[...],
                          preferred_element_type=jnp.float32) * dbc
    y2q_ref[...] = jnp.dot(h, wq_ref[...],
                           preferred_element_type=jnp.float32) * dbc


def _tc_combine_matmul(s0, s1, y1, dbc, b1, W2, W2q):
    return pl.pallas_call(
        _k2_body,
        grid=(_GRID,),
        in_specs=[
            pl.BlockSpec((_BLK, D), lambda i: (i, 0)),
            pl.BlockSpec((_BLK, D), lambda i: (i, 0)),
            pl.BlockSpec((_BLK, D), lambda i: (i, 0)),
            pl.BlockSpec((_BLK, D), lambda i: (i, 0)),
            pl.BlockSpec((1, D), lambda i: (0, 0)),
            pl.BlockSpec((D, D), lambda i: (0, 0)),
            pl.BlockSpec((D, D), lambda i: (0, 0)),
        ],
        out_specs=[
            pl.BlockSpec((_BLK, D), lambda i: (i, 0)),
            pl.BlockSpec((_BLK, D), lambda i: (i, 0)),
            pl.BlockSpec((_BLK, D), lambda i: (i, 0)),
        ],
        out_shape=[
            jax.ShapeDtypeStruct((NPAD, D), jnp.float32),
            jax.ShapeDtypeStruct((NPAD, D), jnp.float32),
            jax.ShapeDtypeStruct((NPAD, D), jnp.float32),
        ],
    )(s0, s1, y1, dbc, b1, W2, W2q)


def _k3_body(s0_ref, s1_ref, y_ref, dbc_ref, b_ref, h_ref):
    s = (s0_ref[...].astype(jnp.float32) + s1_ref[...].astype(jnp.float32))
    h_ref[...] = jnp.maximum(
        dbc_ref[...] * (s + y_ref[...]) + b_ref[...], 0.0)


def _tc_combine(s0, s1, y2, dbc, b2):
    return pl.pallas_call(
        _k3_body,
        grid=(_GRID,),
        in_specs=[
            pl.BlockSpec((_BLK, D), lambda i: (i, 0)),
            pl.BlockSpec((_BLK, D), lambda i: (i, 0)),
            pl.BlockSpec((_BLK, D), lambda i: (i, 0)),
            pl.BlockSpec((_BLK, D), lambda i: (i, 0)),
            pl.BlockSpec((1, D), lambda i: (0, 0)),
        ],
        out_specs=pl.BlockSpec((_BLK, D), lambda i: (i, 0)),
        out_shape=jax.ShapeDtypeStruct((NPAD, D), jnp.float32),
    )(s0, s1, y2, dbc, b2)


def kernel(x, edge_index, W1, b1, W2, b2):
    ei = edge_index.astype(jnp.int32)
    pad = EPAD - N_EDGES
    src = jnp.concatenate(
        [ei[0], jnp.full((pad,), N_NODES, jnp.int32)]).reshape(EPAD // B, B)
    dst = jnp.concatenate(
        [ei[1], jnp.full((pad,), N_NODES, jnp.int32)]).reshape(EPAD // B, B)

    x_pad = jnp.pad(x, ((0, NPAD - N_NODES), (0, 0)))
    ones128 = jnp.ones((B, D), jnp.float32)
    zeros128 = jnp.zeros((B, D), jnp.float32)
    b1r = b1.reshape(1, D)
    b2r = b2.reshape(1, D)

    qperm = jnp.asarray(QPERM)
    W1q = W1[:, qperm]
    W2q = W2[:, qperm]

    degp = _deg_kernel(dst, ones128, zeros128)
    deg0 = degp[:NPAD]
    deg1 = degp[NPAD:]

    y1, y1q, dbc = _tc_scale_matmul(x_pad, deg0, deg1, W1, W1q)

    def unpack_partials(pi):
        pb = lax.bitcast_convert_type(pi, jnp.bfloat16)
        return pb.reshape(NC * NPAD, D)

    s1p = unpack_partials(_scat_kernel(y1q, src, dst, zeros128))
    h1, y2, y2q = _tc_combine_matmul(s1p[:NPAD], s1p[NPAD:], y1, dbc,
                                     b1r, W2, W2q)

    s2p = unpack_partials(_scat_kernel(y2q, src, dst, zeros128))
    h2 = _tc_combine(s2p[:NPAD], s2p[NPAD:], y2, dbc, b2r)

    return jnp.concatenate([h1[:N_NODES], h2[:N_NODES]], axis=1)


# speedup vs baseline: 3.1520x; 3.1520x over previous
import functools

import jax
import jax.numpy as jnp
from jax import lax
from jax.experimental import pallas as pl
from jax.experimental.pallas import tpu as pltpu
from jax.experimental.pallas import tpu_sc as plsc

N_NODES = 10000
N_EDGES = 320000
D = 128

NC = 2
NS = 16
NW = NC * NS

B = 128
NB = 80
EPT = NB * B
EPAD = EPT * NW

NPAD = 10240
RPT = NPAD // NS

_MESH = plsc.VectorSubcoreMesh(core_axis_name="c", subcore_axis_name="s")


def _deg_body(dst_hbm, ones_hbm, zeros_hbm, out_hbm, dstv, onesv, acc):
    c = lax.axis_index("c")
    s = lax.axis_index("s")
    wid = c * NS + s

    pltpu.sync_copy(ones_hbm, onesv)

    for r in range(RPT // B):
        pltpu.sync_copy(zeros_hbm, acc.at[pl.ds(s * RPT + r * B, B)])
    plsc.subcore_barrier()

    def body(jc, carry):
        base = pl.multiple_of(wid * NB + jc * 8, 8)
        pltpu.sync_copy(dst_hbm.at[pl.ds(base, 8)], dstv)
        for jj in range(8):
            pltpu.sync_copy(onesv, acc.at[dstv.at[jj]], add=True)
        return carry

    lax.fori_loop(0, NB // 8, body, 0)
    plsc.subcore_barrier()

    base = c * NPAD + s * RPT
    pltpu.sync_copy(acc.at[pl.ds(s * RPT, RPT)], out_hbm.at[pl.ds(base, RPT)])


@functools.partial(
    pl.kernel,
    mesh=_MESH,
    out_type=jax.ShapeDtypeStruct((NC * NPAD, D), jnp.float32),
    scratch_types=[
        pltpu.VMEM((8, B), jnp.int32),
        pltpu.VMEM((B, D), jnp.float32),
        pltpu.VMEM_SHARED((NPAD, D), jnp.float32),
    ],
)
def _deg_kernel(dst_hbm, ones_hbm, zeros_hbm, out_hbm, dstv, onesv, acc):
    _deg_body(dst_hbm, ones_hbm, zeros_hbm, out_hbm, dstv, onesv, acc)


_CH = 8
_NCHUNK = NB // _CH


def _scat_body(y_hbm, src_hbm, dst_hbm, zeros_hbm, out_hbm,
               srcv, dstv, rows0, rows1, gsem0, gsem1, ssem0, ssem1,
               acc):
    c = lax.axis_index("c")
    s = lax.axis_index("s")
    wid = c * NS + s

    for r in range(RPT // B):
        pltpu.sync_copy(zeros_hbm, acc.at[pl.ds(s * RPT + r * B, B)])
    plsc.subcore_barrier()

    rows = (rows0, rows1)
    gsems = (gsem0, gsem1)
    ssems = (ssem0, ssem1)

    def chunk(jc, carry):
        base = pl.multiple_of(wid * NB + jc * _CH, _CH)
        pltpu.sync_copy(src_hbm.at[pl.ds(base, _CH)], srcv)
        pltpu.sync_copy(dst_hbm.at[pl.ds(base, _CH)], dstv)
        g = [pltpu.async_copy(y_hbm.at[srcv.at[0]], rows[0], gsems[0]),
             pltpu.async_copy(y_hbm.at[srcv.at[1]], rows[1], gsems[1])]
        sc = [None, None]
        for jj in range(_CH):
            b = jj % 2
            g[b].wait()
            sc[b] = pltpu.async_copy(rows[b], acc.at[dstv.at[jj]], ssems[b],
                                     add=True)
            if jj + 2 < _CH:
                sc[b].wait()
                g[b] = pltpu.async_copy(y_hbm.at[srcv.at[jj + 2]], rows[b],
                                        gsems[b])
        sc[0].wait()
        sc[1].wait()
        return carry

    lax.fori_loop(0, _NCHUNK, chunk, 0)
    plsc.subcore_barrier()

    base = c * NPAD + s * RPT
    pltpu.sync_copy(acc.at[pl.ds(s * RPT, RPT)], out_hbm.at[pl.ds(base, RPT)])


@functools.partial(
    pl.kernel,
    mesh=_MESH,
    out_type=jax.ShapeDtypeStruct((NC * NPAD, D), jnp.float32),
    scratch_types=[
        pltpu.VMEM((_CH, B), jnp.int32),
        pltpu.VMEM((_CH, B), jnp.int32),
        pltpu.VMEM((B, D), jnp.float32),
        pltpu.VMEM((B, D), jnp.float32),
        pltpu.SemaphoreType.DMA,
        pltpu.SemaphoreType.DMA,
        pltpu.SemaphoreType.DMA,
        pltpu.SemaphoreType.DMA,
        pltpu.VMEM_SHARED((NPAD, D), jnp.float32),
    ],
)
def _scat_kernel(y_hbm, src_hbm, dst_hbm, zeros_hbm, out_hbm,
                 srcv, dstv, rows0, rows1, gsem0, gsem1, ssem0, ssem1,
                 acc):
    _scat_body(y_hbm, src_hbm, dst_hbm, zeros_hbm, out_hbm,
               srcv, dstv, rows0, rows1, gsem0, gsem1, ssem0, ssem1,
               acc)


_BLK = 512
_GRID = NPAD // _BLK


def _k1_body(x_ref, d0_ref, d1_ref, w_ref, y_ref, dbc_ref):
    deg = d0_ref[...] + d1_ref[...] + 1.0
    d = lax.rsqrt(deg)
    y_ref[...] = jnp.dot(x_ref[...], w_ref[...],
                         preferred_element_type=jnp.float32) * d
    dbc_ref[...] = d


def _tc_scale_matmul(x_pad, deg0, deg1, W1):
    return pl.pallas_call(
        _k1_body,
        grid=(_GRID,),
        in_specs=[
            pl.BlockSpec((_BLK, D), lambda i: (i, 0)),
            pl.BlockSpec((_BLK, D), lambda i: (i, 0)),
            pl.BlockSpec((_BLK, D), lambda i: (i, 0)),
            pl.BlockSpec((D, D), lambda i: (0, 0)),
        ],
        out_specs=[
            pl.BlockSpec((_BLK, D), lambda i: (i, 0)),
            pl.BlockSpec((_BLK, D), lambda i: (i, 0)),
        ],
        out_shape=[
            jax.ShapeDtypeStruct((NPAD, D), jnp.float32),
            jax.ShapeDtypeStruct((NPAD, D), jnp.float32),
        ],
    )(x_pad, deg0, deg1, W1)


def _k2_body(s0_ref, s1_ref, y_ref, dbc_ref, b_ref, w_ref, h_ref, y2_ref):
    dbc = dbc_ref[...]
    h = jnp.maximum(dbc * (s0_ref[...] + s1_ref[...] + y_ref[...])
                    + b_ref[...], 0.0)
    h_ref[...] = h
    y2_ref[...] = jnp.dot(h, w_ref[...],
                          preferred_element_type=jnp.float32) * dbc


def _tc_combine_matmul(s0, s1, y1, dbc, b1, W2):
    return pl.pallas_call(
        _k2_body,
        grid=(_GRID,),
        in_specs=[
            pl.BlockSpec((_BLK, D), lambda i: (i, 0)),
            pl.BlockSpec((_BLK, D), lambda i: (i, 0)),
            pl.BlockSpec((_BLK, D), lambda i: (i, 0)),
            pl.BlockSpec((_BLK, D), lambda i: (i, 0)),
            pl.BlockSpec((1, D), lambda i: (0, 0)),
            pl.BlockSpec((D, D), lambda i: (0, 0)),
        ],
        out_specs=[
            pl.BlockSpec((_BLK, D), lambda i: (i, 0)),
            pl.BlockSpec((_BLK, D), lambda i: (i, 0)),
        ],
        out_shape=[
            jax.ShapeDtypeStruct((NPAD, D), jnp.float32),
            jax.ShapeDtypeStruct((NPAD, D), jnp.float32),
        ],
    )(s0, s1, y1, dbc, b1, W2)


def _k3_body(s0_ref, s1_ref, y_ref, dbc_ref, b_ref, h_ref):
    h_ref[...] = jnp.maximum(
        dbc_ref[...] * (s0_ref[...] + s1_ref[...] + y_ref[...])
        + b_ref[...], 0.0)


def _tc_combine(s0, s1, y2, dbc, b2):
    return pl.pallas_call(
        _k3_body,
        grid=(_GRID,),
        in_specs=[
            pl.BlockSpec((_BLK, D), lambda i: (i, 0)),
            pl.BlockSpec((_BLK, D), lambda i: (i, 0)),
            pl.BlockSpec((_BLK, D), lambda i: (i, 0)),
            pl.BlockSpec((_BLK, D), lambda i: (i, 0)),
            pl.BlockSpec((1, D), lambda i: (0, 0)),
        ],
        out_specs=pl.BlockSpec((_BLK, D), lambda i: (i, 0)),
        out_shape=jax.ShapeDtypeStruct((NPAD, D), jnp.float32),
    )(s0, s1, y2, dbc, b2)


def kernel(x, edge_index, W1, b1, W2, b2):
    ei = edge_index.astype(jnp.int32)
    pad = EPAD - N_EDGES
    src = jnp.concatenate(
        [ei[0], jnp.full((pad,), N_NODES, jnp.int32)]).reshape(EPAD // B, B)
    dst = jnp.concatenate(
        [ei[1], jnp.full((pad,), N_NODES, jnp.int32)]).reshape(EPAD // B, B)

    x_pad = jnp.pad(x, ((0, NPAD - N_NODES), (0, 0)))
    ones128 = jnp.ones((B, D), jnp.float32)
    zeros128 = jnp.zeros((B, D), jnp.float32)
    b1r = b1.reshape(1, D)
    b2r = b2.reshape(1, D)

    degp = _deg_kernel(dst, ones128, zeros128)
    deg0 = degp[:NPAD]
    deg1 = degp[NPAD:]

    y1, dbc = _tc_scale_matmul(x_pad, deg0, deg1, W1)

    s1p = _scat_kernel(y1, src, dst, zeros128)
    h1, y2 = _tc_combine_matmul(s1p[:NPAD], s1p[NPAD:], y1, dbc, b1r, W2)

    s2p = _scat_kernel(y2, src, dst, zeros128)
    h2 = _tc_combine(s2p[:NPAD], s2p[NPAD:], y2, dbc, b2r)

    return jnp.concatenate([h1[:N_NODES], h2[:N_NODES]], axis=1)
